# baseline (speedup 1.0000x reference)
import functools
import math

import jax
import jax.numpy as jnp
from jax import lax
from jax.experimental import pallas as pl
from jax.experimental.pallas import tpu as pltpu
from jax.experimental.pallas import tpu_sc as plsc

_N = 10000
_D = 128
_E = 160000

_NC = 2
_NS = 16
_CHUNK = 128
_CHUNKS = 40
_EPAD = _NC * _NS * _CHUNKS * _CHUNK
_NPAD = 10240
_RPT = _NPAD // _NS
_RHALF = _RPT // 2


def _seg_sum_parts(x, srcs, dsts, zeros):
    mesh = plsc.VectorSubcoreMesh(core_axis_name="c", subcore_axis_name="s")

    @functools.partial(
        pl.kernel,
        mesh=mesh,
        out_type=jax.ShapeDtypeStruct((_NC, _NPAD, _D), jnp.float32),
        scratch_types=[
            pltpu.VMEM((_CHUNKS, _CHUNK), jnp.int32),
            pltpu.VMEM((_CHUNKS, _CHUNK), jnp.int32),
            pltpu.VMEM((_CHUNK, _D), jnp.float32),
            pltpu.VMEM((_CHUNK, _D), jnp.float32),
            pltpu.VMEM_SHARED((_NPAD, _D), jnp.float32),
            pltpu.SemaphoreType.DMA,
            pltpu.SemaphoreType.DMA,
        ],
    )
    def seg(x_hbm, src_hbm, dst_hbm, zero_hbm, out_hbm,
            src_v, dst_v, rows0, rows1, acc, sem0, sem1):
        c = lax.axis_index("c")
        s = lax.axis_index("s")
        pltpu.sync_copy(src_hbm.at[c, s], src_v)
        pltpu.sync_copy(dst_hbm.at[c, s], dst_v)
        base = s * _RPT
        pltpu.sync_copy(zero_hbm.at[pl.ds(base, _RPT)],
                        acc.at[pl.ds(base, _RPT)])
        plsc.subcore_barrier()

        pltpu.async_copy(x_hbm.at[src_v.at[0]], rows0, sem0)
        pltpu.async_copy(x_hbm.at[src_v.at[1]], rows1, sem1)

        def _wait(buf, sem):
            pltpu.make_async_copy(x_hbm.at[pl.ds(0, _CHUNK)], buf, sem).wait()

        def body(jj, carry):
            j0 = jj * 2
            j1 = j0 + 1
            _wait(rows0, sem0)
            pltpu.sync_copy(rows0, acc.at[dst_v.at[j0]], add=True)
            pltpu.async_copy(
                x_hbm.at[src_v.at[jnp.minimum(j0 + 2, _CHUNKS - 1)]],
                rows0, sem0)
            _wait(rows1, sem1)
            pltpu.sync_copy(rows1, acc.at[dst_v.at[j1]], add=True)
            pltpu.async_copy(
                x_hbm.at[src_v.at[jnp.minimum(j1 + 2, _CHUNKS - 1)]],
                rows1, sem1)
            return carry

        lax.fori_loop(0, _CHUNKS // 2, body, 0)
        _wait(rows0, sem0)
        _wait(rows1, sem1)
        plsc.subcore_barrier()
        pltpu.sync_copy(acc.at[pl.ds(base, _RPT)],
                        out_hbm.at[c, pl.ds(base, _RPT)])

    return seg(x, srcs, dsts, zeros)


_BLK = 2000


def _mlp_stats(x, agg_a, agg_b, w1, b1, w2, b2):
    n, d = x.shape
    fo = w2.shape[1]
    steps = n // _BLK

    def kern(x_ref, a_ref, b_ref, w1_ref, b1_ref, w2_ref, b2_ref,
             h_ref, st_ref):
        i = pl.program_id(0)
        xx = x_ref[...] + a_ref[...] + b_ref[...]
        h1 = jnp.maximum(
            jnp.dot(xx, w1_ref[...], preferred_element_type=jnp.float32)
            + b1_ref[...], 0.0)
        h2 = jnp.maximum(
            jnp.dot(h1, w2_ref[...], preferred_element_type=jnp.float32)
            + b2_ref[...], 0.0)
        h_ref[...] = h2
        st = jnp.concatenate(
            [jnp.sum(h2, axis=0, keepdims=True),
             jnp.sum(h2 * h2, axis=0, keepdims=True)], axis=0)

        @pl.when(i == 0)
        def _():
            st_ref[...] = st

        @pl.when(i != 0)
        def _():
            st_ref[...] = st_ref[...] + st

    fixed = lambda shape: pl.BlockSpec(shape, lambda i: (0, 0))
    return pl.pallas_call(
        kern,
        grid=(steps,),
        in_specs=[
            pl.BlockSpec((_BLK, d), lambda i: (i, 0)),
            pl.BlockSpec((_BLK, d), lambda i: (i, 0)),
            pl.BlockSpec((_BLK, d), lambda i: (i, 0)),
            fixed(w1.shape), fixed(b1.shape), fixed(w2.shape), fixed(b2.shape),
        ],
        out_specs=[
            pl.BlockSpec((_BLK, fo), lambda i: (i, 0)),
            fixed((2, fo)),
        ],
        out_shape=[
            jax.ShapeDtypeStruct((n, fo), jnp.float32),
            jax.ShapeDtypeStruct((2, fo), jnp.float32),
        ],
    )(x, agg_a, agg_b, w1, b1, w2, b2)


def _bn_qkv(h, st, g, b, wq, bq, wk, bk, wv, bv, prec=None):
    n, d = h.shape

    def kern(h_ref, st_ref, g_ref, b_ref, wq_ref, bq_ref, wk_ref, bk_ref,
             wv_ref, bv_ref, q_ref, k_ref, v_ref):
        stv = st_ref[...]
        mean = stv[0:1, :] * (1.0 / _N)
        var = stv[1:2, :] * (1.0 / _N) - mean * mean
        scale = g_ref[...] * lax.rsqrt(var + 1e-5)
        shift = b_ref[...] - mean * scale
        hn = h_ref[...] * scale + shift
        q_ref[...] = jnp.dot(hn, wq_ref[...], precision=prec,
                             preferred_element_type=jnp.float32) + bq_ref[...]
        k_ref[...] = jnp.dot(hn, wk_ref[...], precision=prec,
                             preferred_element_type=jnp.float32) + bk_ref[...]
        v_ref[...] = jnp.dot(hn, wv_ref[...], precision=prec,
                             preferred_element_type=jnp.float32) + bv_ref[...]

    fixed = lambda shape: pl.BlockSpec(shape, lambda i: (0, 0))
    blk = pl.BlockSpec((_BLK, d), lambda i: (i, 0))
    return pl.pallas_call(
        kern,
        grid=(n // _BLK,),
        in_specs=[blk, fixed((2, d)), fixed((1, d)), fixed((1, d)),
                  fixed((d, d)), fixed((1, d)), fixed((d, d)), fixed((1, d)),
                  fixed((d, d)), fixed((1, d))],
        out_specs=[blk, blk, blk],
        out_shape=[jax.ShapeDtypeStruct((n, d), jnp.float32)] * 3,
    )(h, st, g, b, wq, bq, wk, bk, wv, bv)


def _attn(q, k, v, sm_scale, prec=None):
    n, d = q.shape
    bq = 200

    def kern(q_ref, k_ref, v_ref, o_ref):
        s = lax.dot_general(q_ref[...], k_ref[...],
                            (((1,), (1,)), ((), ())), precision=prec,
                            preferred_element_type=jnp.float32) * sm_scale
        m = jnp.max(s, axis=1, keepdims=True)
        p = jnp.exp(s - m)
        p = p / jnp.sum(p, axis=1, keepdims=True)
        o_ref[...] = lax.dot_general(p, v_ref[...], (((1,), (0,)), ((), ())),
                                     precision=prec,
                                     preferred_element_type=jnp.float32)

    fixed = pl.BlockSpec((n, d), lambda i: (0, 0))
    blk = pl.BlockSpec((bq, d), lambda i: (i, 0))
    return pl.pallas_call(
        kern,
        grid=(n // bq,),
        in_specs=[blk, fixed, fixed],
        out_specs=blk,
        out_shape=jax.ShapeDtypeStruct((n, d), jnp.float32),
    )(q, k, v)


def _pad_cols(a, width):
    return jnp.pad(a, ((0, 0), (0, width - a.shape[1])))


def _row(a, width=None):
    if width is not None:
        a = jnp.pad(a, (0, width - a.shape[0]))
    return a.reshape(1, -1)


def kernel(x, edge_index, n1_w1, n1_b1, n1_w2, n1_b2, bn1_g, bn1_b,
           a1_wk, a1_bk, a1_wq, a1_bq, a1_wv, a1_bv,
           n2_w1, n2_b1, n2_w2, n2_b2, bn2_g, bn2_b,
           a2_wk, a2_bk, a2_wq, a2_bq, a2_wv, a2_bv):
    src, dst = edge_index[0], edge_index[1]
    pad = _EPAD - _E
    srcs = jnp.concatenate([src, jnp.zeros((pad,), src.dtype)])
    srcs = srcs.reshape(_NC, _NS, _CHUNKS, _CHUNK)
    dsts = jnp.concatenate([dst, jnp.full((pad,), _N, dst.dtype)])
    dsts = dsts.reshape(_NC, _NS, _CHUNKS, _CHUNK)
    zeros = jnp.zeros((_NPAD, _D), jnp.float32)

    parts = _seg_sum_parts(x, srcs, dsts, zeros)
    h1, st1 = _mlp_stats(x, parts[0, :_N], parts[1, :_N],
                         n1_w1, _row(n1_b1), n1_w2, _row(n1_b2))
    q1, k1, v1 = _bn_qkv(h1, st1, _row(bn1_g), _row(bn1_b),
                         a1_wq, _row(a1_bq), a1_wk, _row(a1_bk),
                         a1_wv, _row(a1_bv))
    hA = _attn(q1, k1, v1, 1.0 / math.sqrt(float(_D)))

    F2 = 8
    hp = lax.Precision.HIGHEST
    parts2 = _seg_sum_parts(hA, srcs, dsts, zeros)
    w2p = _pad_cols(n2_w2, F2)
    h2, st2 = _mlp_stats(hA, parts2[0, :_N], parts2[1, :_N],
                         n2_w1, _row(n2_b1), w2p, _row(n2_b2, F2))
    q2, k2, v2 = _bn_qkv(h2, st2, _row(bn2_g, F2), _row(bn2_b, F2),
                         _pad_cols(jnp.pad(a2_wq, ((0, F2 - 2), (0, 0))), F2),
                         _row(a2_bq, F2),
                         _pad_cols(jnp.pad(a2_wk, ((0, F2 - 2), (0, 0))), F2),
                         _row(a2_bk, F2),
                         _pad_cols(jnp.pad(a2_wv, ((0, F2 - 2), (0, 0))), F2),
                         _row(a2_bv, F2), prec=hp)
    out = _attn(q2, k2, v2, 1.0 / math.sqrt(2.0), prec=hp)
    return out[:, :2]


# speedup vs baseline: 1.8854x; 1.8854x over previous
import functools
import math

import jax
import jax.numpy as jnp
from jax import lax
from jax.experimental import pallas as pl
from jax.experimental.pallas import tpu as pltpu
from jax.experimental.pallas import tpu_sc as plsc

_N = 10000
_D = 128
_E = 160000

_NC = 2
_NS = 16
_CHUNK = 128
_CHUNKS = 40
_EPAD = _NC * _NS * _CHUNKS * _CHUNK
_NPAD = 10240
_RPT = _NPAD // _NS
_RHALF = _RPT // 2


def _seg_sum_parts(x, srcs, dsts, zeros):
    mesh = plsc.VectorSubcoreMesh(core_axis_name="c", subcore_axis_name="s")

    @functools.partial(
        pl.kernel,
        mesh=mesh,
        out_type=jax.ShapeDtypeStruct((_NC, _NPAD, _D), jnp.float32),
        scratch_types=[
            pltpu.VMEM((_CHUNKS, _CHUNK), jnp.int32),
            pltpu.VMEM((_CHUNKS, _CHUNK), jnp.int32),
            pltpu.VMEM((_CHUNK, _D), jnp.float32),
            pltpu.VMEM((_CHUNK, _D), jnp.float32),
            pltpu.VMEM_SHARED((_NPAD, _D), jnp.float32),
            pltpu.SemaphoreType.DMA,
            pltpu.SemaphoreType.DMA,
        ],
    )
    def seg(x_hbm, src_hbm, dst_hbm, zero_hbm, out_hbm,
            src_v, dst_v, rows0, rows1, acc, sem0, sem1):
        c = lax.axis_index("c")
        s = lax.axis_index("s")
        pltpu.sync_copy(src_hbm.at[c, s], src_v)
        pltpu.sync_copy(dst_hbm.at[c, s], dst_v)
        base = s * _RPT
        pltpu.sync_copy(zero_hbm.at[pl.ds(base, _RPT)],
                        acc.at[pl.ds(base, _RPT)])
        plsc.subcore_barrier()

        pltpu.async_copy(x_hbm.at[src_v.at[0]], rows0, sem0)
        pltpu.async_copy(x_hbm.at[src_v.at[1]], rows1, sem1)

        def _wait(buf, sem):
            pltpu.make_async_copy(x_hbm.at[pl.ds(0, _CHUNK)], buf, sem).wait()

        def body(jj, carry):
            j0 = jj * 2
            j1 = j0 + 1
            _wait(rows0, sem0)
            pltpu.sync_copy(rows0, acc.at[dst_v.at[j0]], add=True)
            pltpu.async_copy(
                x_hbm.at[src_v.at[jnp.minimum(j0 + 2, _CHUNKS - 1)]],
                rows0, sem0)
            _wait(rows1, sem1)
            pltpu.sync_copy(rows1, acc.at[dst_v.at[j1]], add=True)
            pltpu.async_copy(
                x_hbm.at[src_v.at[jnp.minimum(j1 + 2, _CHUNKS - 1)]],
                rows1, sem1)
            return carry

        lax.fori_loop(0, _CHUNKS // 2, body, 0)
        _wait(rows0, sem0)
        _wait(rows1, sem1)
        plsc.subcore_barrier()
        pltpu.sync_copy(acc.at[pl.ds(base, _RPT)],
                        out_hbm.at[c, pl.ds(base, _RPT)])

    return seg(x, srcs, dsts, zeros)


_BLK = 2000


def _mlp_stats(x, agg_a, agg_b, w1, b1, w2, b2):
    n, d = x.shape
    fo = w2.shape[1]
    steps = n // _BLK

    def kern(x_ref, a_ref, b_ref, w1_ref, b1_ref, w2_ref, b2_ref,
             h_ref, st_ref):
        i = pl.program_id(0)
        xx = x_ref[...] + a_ref[...] + b_ref[...]
        h1 = jnp.maximum(
            jnp.dot(xx, w1_ref[...], preferred_element_type=jnp.float32)
            + b1_ref[...], 0.0)
        h2 = jnp.maximum(
            jnp.dot(h1, w2_ref[...], preferred_element_type=jnp.float32)
            + b2_ref[...], 0.0)
        h_ref[...] = h2
        st = jnp.concatenate(
            [jnp.sum(h2, axis=0, keepdims=True),
             jnp.sum(h2 * h2, axis=0, keepdims=True)], axis=0)

        @pl.when(i == 0)
        def _():
            st_ref[...] = st

        @pl.when(i != 0)
        def _():
            st_ref[...] = st_ref[...] + st

    fixed = lambda shape: pl.BlockSpec(shape, lambda i: (0, 0))
    return pl.pallas_call(
        kern,
        grid=(steps,),
        in_specs=[
            pl.BlockSpec((_BLK, d), lambda i: (i, 0)),
            pl.BlockSpec((_BLK, d), lambda i: (i, 0)),
            pl.BlockSpec((_BLK, d), lambda i: (i, 0)),
            fixed(w1.shape), fixed(b1.shape), fixed(w2.shape), fixed(b2.shape),
        ],
        out_specs=[
            pl.BlockSpec((_BLK, fo), lambda i: (i, 0)),
            fixed((2, fo)),
        ],
        out_shape=[
            jax.ShapeDtypeStruct((n, fo), jnp.float32),
            jax.ShapeDtypeStruct((2, fo), jnp.float32),
        ],
    )(x, agg_a, agg_b, w1, b1, w2, b2)


def _bn_qkv(h, st, g, b, wq, bq, wk, bk, wv, bv, prec=None):
    n, d = h.shape

    def kern(h_ref, st_ref, g_ref, b_ref, wq_ref, bq_ref, wk_ref, bk_ref,
             wv_ref, bv_ref, q_ref, k_ref, v_ref):
        stv = st_ref[...]
        mean = stv[0:1, :] * (1.0 / _N)
        var = stv[1:2, :] * (1.0 / _N) - mean * mean
        scale = g_ref[...] * lax.rsqrt(var + 1e-5)
        shift = b_ref[...] - mean * scale
        hn = h_ref[...] * scale + shift
        q_ref[...] = jnp.dot(hn, wq_ref[...], precision=prec,
                             preferred_element_type=jnp.float32) + bq_ref[...]
        k_ref[...] = jnp.dot(hn, wk_ref[...], precision=prec,
                             preferred_element_type=jnp.float32) + bk_ref[...]
        v_ref[...] = jnp.dot(hn, wv_ref[...], precision=prec,
                             preferred_element_type=jnp.float32) + bv_ref[...]

    fixed = lambda shape: pl.BlockSpec(shape, lambda i: (0, 0))
    blk = pl.BlockSpec((_BLK, d), lambda i: (i, 0))
    return pl.pallas_call(
        kern,
        grid=(n // _BLK,),
        in_specs=[blk, fixed((2, d)), fixed((1, d)), fixed((1, d)),
                  fixed((d, d)), fixed((1, d)), fixed((d, d)), fixed((1, d)),
                  fixed((d, d)), fixed((1, d))],
        out_specs=[blk, blk, blk],
        out_shape=[jax.ShapeDtypeStruct((n, d), jnp.float32)] * 3,
    )(h, st, g, b, wq, bq, wk, bk, wv, bv)


def _attn(q, k, v, sm_scale, prec=None):
    n, d = q.shape
    bq = 200

    def kern(q_ref, k_ref, v_ref, o_ref):
        s = lax.dot_general(q_ref[...], k_ref[...],
                            (((1,), (1,)), ((), ())), precision=prec,
                            preferred_element_type=jnp.float32) * sm_scale
        m = jnp.max(s, axis=1, keepdims=True)
        p = jnp.exp(s - m)
        p = p / jnp.sum(p, axis=1, keepdims=True)
        o_ref[...] = lax.dot_general(p, v_ref[...], (((1,), (0,)), ((), ())),
                                     precision=prec,
                                     preferred_element_type=jnp.float32)

    fixed = pl.BlockSpec((n, d), lambda i: (0, 0))
    blk = pl.BlockSpec((bq, d), lambda i: (i, 0))
    return pl.pallas_call(
        kern,
        grid=(n // bq,),
        in_specs=[blk, fixed, fixed],
        out_specs=blk,
        out_shape=jax.ShapeDtypeStruct((n, d), jnp.float32),
    )(q, k, v)


def _pad_cols(a, width):
    return jnp.pad(a, ((0, 0), (0, width - a.shape[1])))


def _row(a, width=None):
    if width is not None:
        a = jnp.pad(a, (0, width - a.shape[0]))
    return a.reshape(1, -1)


def kernel(x, edge_index, n1_w1, n1_b1, n1_w2, n1_b2, bn1_g, bn1_b,
           a1_wk, a1_bk, a1_wq, a1_bq, a1_wv, a1_bv,
           n2_w1, n2_b1, n2_w2, n2_b2, bn2_g, bn2_b,
           a2_wk, a2_bk, a2_wq, a2_bq, a2_wv, a2_bv):
    src, dst = edge_index[0], edge_index[1]
    pad = _EPAD - _E
    srcs = jnp.concatenate([src, jnp.zeros((pad,), src.dtype)])
    srcs = srcs.reshape(_NC, _NS, _CHUNKS, _CHUNK)
    dsts = jnp.concatenate([dst, jnp.full((pad,), _N, dst.dtype)])
    dsts = dsts.reshape(_NC, _NS, _CHUNKS, _CHUNK)
    zeros = jnp.zeros((_NPAD, _D), jnp.float32)

    parts = _seg_sum_parts(x, srcs, dsts, zeros)
    h1, st1 = _mlp_stats(x, parts[0, :_N], parts[1, :_N],
                         n1_w1, _row(n1_b1), n1_w2, _row(n1_b2))
    q1, k1, v1 = _bn_qkv(h1, st1, _row(bn1_g), _row(bn1_b),
                         a1_wq, _row(a1_bq), a1_wk, _row(a1_bk),
                         a1_wv, _row(a1_bv))
    hA = _attn(q1, k1, v1, 1.0 / math.sqrt(float(_D)))

    F2 = 8
    hp = lax.Precision.HIGHEST
    parts2 = _seg_sum_parts(hA, srcs, dsts, zeros)
    w2p = _pad_cols(n2_w2, F2)
    h2, st2 = _mlp_stats(hA, parts2[0, :_N], parts2[1, :_N],
                         n2_w1, _row(n2_b1), w2p, _row(n2_b2, F2))
    q2, k2, v2 = _bn_qkv(h2, st2, _row(bn2_g, F2), _row(bn2_b, F2),
                         _pad_cols(jnp.pad(a2_wq, ((0, F2 - 2), (0, 0))), F2),
                         _row(a2_bq, F2),
                         _pad_cols(jnp.pad(a2_wk, ((0, F2 - 2), (0, 0))), F2),
                         _row(a2_bk, F2),
                         _pad_cols(jnp.pad(a2_wv, ((0, F2 - 2), (0, 0))), F2),
                         _row(a2_bv, F2), prec=hp)
    out = _attn(q2, k2, v2, 1.0 / math.sqrt(2.0))
    return out[:, :2]
